# baseline (speedup 1.0000x reference)
import functools

import jax
import jax.numpy as jnp
from jax import lax
from jax.experimental import pallas as pl
from jax.experimental.pallas import tpu as pltpu
from jax.experimental.pallas import tpu_sc as plsc

H = 32
N_NODES = 10000
N_VERTS = 50000
E = 320000

NC = 2
NS = 16
NW = NC * NS
EW = E // NW
CKG = 1000
NCHG = EW // CKG

VH = 25088
ACC_ROWS = VH + 16
TROWS = VH // NS
ESUB = E // NS
CKS = 800
NCH_S = ESUB // CKS

BR = 5000
BE = 8000


def _rows_block(n):
    return BE if n % BE == 0 else BR


def _bn(y):
    m = jnp.mean(y, axis=0, keepdims=True)
    v = jnp.mean((y - m) * (y - m), axis=0, keepdims=True)
    return (y - m) * lax.rsqrt(v + 1e-5)


def _gdf(x, start, stop):
    step = (stop - start) / 7.0
    c = start + step * lax.broadcasted_iota(jnp.int32, (1, 8), 1).astype(jnp.float32)
    d = x - c
    return jnp.exp(-(d * d) / (step * step))


def _silu(x):
    return x * jax.nn.sigmoid(x)


def _chem_body(gx_ref, np_ref, wc1, bc1, wc2, bc2, wp, chem_ref, tp_ref):
    gx = gx_ref[...]
    h = _silu(_bn(jnp.dot(gx, wc1[...], preferred_element_type=jnp.float32) + bc1[...]))
    chem_ref[...] = _bn(jnp.dot(h, wc2[...], preferred_element_type=jnp.float32) + bc2[...])
    proj = jnp.dot(gx, wp[...], preferred_element_type=jnp.float32)
    pad = jnp.zeros((N_NODES, 13), jnp.float32)
    tp_ref[...] = jnp.concatenate([proj, np_ref[...], pad], axis=1)


def _chem_stage(graph_x, node_pos, wc1, bc1, wc2, bc2, wp):
    return pl.pallas_call(
        _chem_body,
        out_shape=(
            jax.ShapeDtypeStruct((N_NODES, H), jnp.float32),
            jax.ShapeDtypeStruct((N_NODES, 48), jnp.float32),
        ),
    )(graph_x, node_pos, wc1, bc1, wc2, bc2, wp)


def _lin1_body(x_ref, w1, b1, y_ref, st_ref):
    i = pl.program_id(0)
    y = jnp.dot(x_ref[...], w1[...], preferred_element_type=jnp.float32) + b1[...]
    y_ref[...] = y

    @pl.when(i == 0)
    def _():
        st_ref[...] = jnp.zeros_like(st_ref)

    st_ref[0:1, :] += jnp.sum(y, axis=0, keepdims=True)
    st_ref[1:2, :] += jnp.sum(y * y, axis=0, keepdims=True)


def _lin1_stage(x, w1, b1):
    n, fi = x.shape
    return pl.pallas_call(
        _lin1_body,
        grid=(n // BR,),
        in_specs=[
            pl.BlockSpec((BR, fi), lambda i: (i, 0)),
            pl.BlockSpec((fi, H), lambda i: (0, 0)),
            pl.BlockSpec((1, H), lambda i: (0, 0)),
        ],
        out_specs=[
            pl.BlockSpec((BR, H), lambda i: (i, 0)),
            pl.BlockSpec((8, H), lambda i: (0, 0)),
        ],
        out_shape=(
            jax.ShapeDtypeStruct((n, H), jnp.float32),
            jax.ShapeDtypeStruct((8, H), jnp.float32),
        ),
    )(x, w1, b1)


def _st_moments(st_ref, n):
    mu = st_ref[0:1, :] * (1.0 / n)
    var = st_ref[1:2, :] * (1.0 / n) - mu * mu
    return mu, lax.rsqrt(var + 1e-5)


def _mid_body(n, y_ref, st1, w2, b2, st_ref):
    i = pl.program_id(0)
    mu1, is1 = _st_moments(st1, n)
    h = _silu((y_ref[...] - mu1) * is1)
    z = jnp.dot(h, w2[...], preferred_element_type=jnp.float32) + b2[...]

    @pl.when(i == 0)
    def _():
        st_ref[...] = jnp.zeros_like(st_ref)

    st_ref[0:1, :] += jnp.sum(z, axis=0, keepdims=True)
    st_ref[1:2, :] += jnp.sum(z * z, axis=0, keepdims=True)


def _mid_stage(y, st1, w2, b2):
    n, _ = y.shape
    wo = w2.shape[1]
    br = _rows_block(n)
    return pl.pallas_call(
        functools.partial(_mid_body, n),
        grid=(n // br,),
        in_specs=[
            pl.BlockSpec((br, H), lambda i: (i, 0)),
            pl.BlockSpec((8, H), lambda i: (0, 0)),
            pl.BlockSpec((H, wo), lambda i: (0, 0)),
            pl.BlockSpec((1, wo), lambda i: (0, 0)),
        ],
        out_specs=pl.BlockSpec((8, wo), lambda i: (0, 0)),
        out_shape=jax.ShapeDtypeStruct((8, wo), jnp.float32),
    )(y, st1, w2, b2)


def _apply_body(n, y_ref, st1, st2, w2, b2, out_ref):
    mu1, is1 = _st_moments(st1, n)
    mu2, is2 = _st_moments(st2, n)
    h = _silu((y_ref[...] - mu1) * is1)
    z = jnp.dot(h, w2[...] * is2, preferred_element_type=jnp.float32)
    out_ref[...] = z + (b2[...] - mu2) * is2


def _apply_stage(y, st1, st2, w2, b2):
    n, _ = y.shape
    br = _rows_block(n)
    return pl.pallas_call(
        functools.partial(_apply_body, n),
        grid=(n // br,),
        in_specs=[
            pl.BlockSpec((br, H), lambda i: (i, 0)),
            pl.BlockSpec((8, H), lambda i: (0, 0)),
            pl.BlockSpec((8, H), lambda i: (0, 0)),
            pl.BlockSpec((H, H), lambda i: (0, 0)),
            pl.BlockSpec((1, H), lambda i: (0, 0)),
        ],
        out_specs=pl.BlockSpec((br, H), lambda i: (i, 0)),
        out_shape=jax.ShapeDtypeStruct((n, H), jnp.float32),
    )(y, st1, st2, w2, b2)


def _gather_sc(tp, tvv, ind, vids):
    mesh = plsc.VectorSubcoreMesh(core_axis_name="c", subcore_axis_name="s")

    @functools.partial(
        pl.kernel,
        out_type=(
            jax.ShapeDtypeStruct((E, 48), jnp.float32),
            jax.ShapeDtypeStruct((E, 16), jnp.float32),
        ),
        mesh=mesh,
        scratch_types=[
            pltpu.VMEM((CKG,), jnp.int32),
            pltpu.VMEM((CKG,), jnp.int32),
            pltpu.VMEM((CKG, 48), jnp.float32),
            pltpu.VMEM((CKG, 16), jnp.float32),
            pltpu.SemaphoreType.DMA,
            pltpu.SemaphoreType.DMA,
        ],
        compiler_params=pltpu.CompilerParams(use_tc_tiling_on_sc=False),
    )
    def k(tp_hbm, tvv_hbm, ind_hbm, vids_hbm, oa_hbm, oc_hbm,
          idx1_v, idx2_v, ra_v, rc_v, sem_a, sem_c):
        wid = lax.axis_index("s") * NC + lax.axis_index("c")

        def chunk(i, carry):
            base = pl.multiple_of(wid * EW + i * CKG, 8)
            pltpu.sync_copy(ind_hbm.at[pl.ds(base, CKG)], idx1_v)
            pltpu.sync_copy(vids_hbm.at[pl.ds(base, CKG)], idx2_v)
            da = pltpu.async_copy(tp_hbm.at[idx1_v], ra_v, sem_a)
            dc = pltpu.async_copy(tvv_hbm.at[idx2_v], rc_v, sem_c)
            da.wait()
            dc.wait()
            pltpu.sync_copy(ra_v, oa_hbm.at[pl.ds(base, CKG)])
            pltpu.sync_copy(rc_v, oc_hbm.at[pl.ds(base, CKG)])
            return carry

        lax.fori_loop(0, NCHG, chunk, 0)

    return k(tp, tvv, ind, vids)


def _edge_a_body(ga_ref, gvv_ref, d_ref, w1da, b1, y_ref, st_ref):
    i = pl.program_id(0)
    p8 = ga_ref[:, 32:40]
    v8 = gvv_ref[:, 0:8]
    n8 = gvv_ref[:, 8:16]
    dlt = p8 - v8
    ones8 = jnp.full((8, 1), 1.0, jnp.float32)
    nrm2 = jnp.dot(dlt * dlt, ones8, preferred_element_type=jnp.float32)
    dot = jnp.dot(dlt * n8, ones8, preferred_element_type=jnp.float32)
    ang = dot * lax.rsqrt(nrm2)
    gda = jnp.concatenate([_gdf(d_ref[...], 0.0, 8.0), _gdf(ang, -1.0, 1.0)],
                          axis=1)
    y = (ga_ref[:, 0:32]
         + jnp.dot(gda, w1da[...], preferred_element_type=jnp.float32)
         + b1[...])
    y_ref[...] = y

    @pl.when(i == 0)
    def _():
        st_ref[...] = jnp.zeros_like(st_ref)

    st_ref[0:1, :] += jnp.sum(y, axis=0, keepdims=True)
    st_ref[1:2, :] += jnp.sum(y * y, axis=0, keepdims=True)


def _edge_a(gath_p, gath_vv, dist2d, w1da, b1):
    return pl.pallas_call(
        _edge_a_body,
        grid=(E // BE,),
        in_specs=[
            pl.BlockSpec((BE, 48), lambda i: (i, 0)),
            pl.BlockSpec((BE, 16), lambda i: (i, 0)),
            pl.BlockSpec((BE, 1), lambda i: (i, 0)),
            pl.BlockSpec((16, H), lambda i: (0, 0)),
            pl.BlockSpec((1, H), lambda i: (0, 0)),
        ],
        out_specs=[
            pl.BlockSpec((BE, H), lambda i: (i, 0)),
            pl.BlockSpec((8, H), lambda i: (0, 0)),
        ],
        out_shape=(
            jax.ShapeDtypeStruct((E, H), jnp.float32),
            jax.ShapeDtypeStruct((8, H), jnp.float32),
        ),
    )(gath_p, gath_vv, dist2d, w1da, b1)


def _edge_c_body(y_ref, st1, st2, w2f, b2f, w2c, b2c, u_ref):
    mu1, is1 = _st_moments(st1, E)
    mu2, is2 = _st_moments(st2, E)
    mu2f, mu2c = mu2[:, 0:H], mu2[:, H:2 * H]
    is2f, is2c = is2[:, 0:H], is2[:, H:2 * H]
    h = _silu((y_ref[...] - mu1) * is1)
    zf = (jnp.dot(h, w2f[...] * is2f, preferred_element_type=jnp.float32)
          + (b2f[...] - mu2f) * is2f)
    zc = (jnp.dot(h, w2c[...] * is2c, preferred_element_type=jnp.float32)
          + (b2c[...] - mu2c) * is2c)
    gate = jax.nn.sigmoid(zf)
    sp = jnp.maximum(zc, 0.0) + jnp.log1p(jnp.exp(-jnp.abs(zc)))
    u_ref[...] = gate * sp


def _edge_c(y, st1, st2, w2f, b2f, w2c, b2c):
    return pl.pallas_call(
        _edge_c_body,
        grid=(E // BE,),
        in_specs=[
            pl.BlockSpec((BE, H), lambda i: (i, 0)),
            pl.BlockSpec((8, H), lambda i: (0, 0)),
            pl.BlockSpec((8, 2 * H), lambda i: (0, 0)),
            pl.BlockSpec((H, H), lambda i: (0, 0)),
            pl.BlockSpec((1, H), lambda i: (0, 0)),
            pl.BlockSpec((H, H), lambda i: (0, 0)),
            pl.BlockSpec((1, H), lambda i: (0, 0)),
        ],
        out_specs=pl.BlockSpec((BE, H), lambda i: (i, 0)),
        out_shape=jax.ShapeDtypeStruct((E, H), jnp.float32),
    )(y, st1, st2, w2f, b2f, w2c, b2c)


def _scatter_sc(u, vids):
    mesh = plsc.VectorSubcoreMesh(core_axis_name="c", subcore_axis_name="s")

    @functools.partial(
        pl.kernel,
        out_type=jax.ShapeDtypeStruct((NC * VH, H), jnp.float32),
        mesh=mesh,
        scratch_types=[
            pltpu.VMEM((CKS, H), jnp.float32),
            pltpu.VMEM((CKS,), jnp.int32),
            pltpu.VMEM_SHARED((ACC_ROWS, H), jnp.float32),
        ],
        compiler_params=pltpu.CompilerParams(use_tc_tiling_on_sc=False),
    )
    def k(u_hbm, vids_hbm, out_hbm, val_v, idx_v, acc_sh):
        cid = lax.axis_index("c")
        sid = lax.axis_index("s")
        r0 = sid * TROWS
        vbase = cid * VH

        def zbody(r, carry):
            z16 = jnp.zeros((16,), jnp.float32)
            val_v[r, 0:16] = z16
            val_v[r, 16:32] = z16
            return carry

        lax.fori_loop(0, CKS, zbody, 0)
        pltpu.sync_copy(val_v, acc_sh.at[pl.ds(r0, CKS)])
        pltpu.sync_copy(val_v.at[pl.ds(0, TROWS - CKS)],
                        acc_sh.at[pl.ds(r0 + CKS, TROWS - CKS)])

        @pl.when(sid == 0)
        def _():
            pltpu.sync_copy(val_v.at[pl.ds(0, 16)], acc_sh.at[pl.ds(VH, 16)])

        plsc.subcore_barrier()

        def chunk(i, carry):
            base = pl.multiple_of(sid * ESUB + i * CKS, 8)
            pltpu.sync_copy(u_hbm.at[pl.ds(base, CKS)], val_v)
            pltpu.sync_copy(vids_hbm.at[pl.ds(base, CKS)], idx_v)

            def xbody(j, c2):
                v = idx_v[pl.ds(j * 16, 16)] - vbase
                ok = (v >= 0) & (v < VH)
                idx_v[pl.ds(j * 16, 16)] = jnp.where(ok, v, VH)
                return c2

            lax.fori_loop(0, CKS // 16, xbody, 0)
            pltpu.sync_copy(val_v, acc_sh.at[idx_v], add=True)
            return carry

        lax.fori_loop(0, NCH_S, chunk, 0)
        plsc.subcore_barrier()

        o0 = vbase + r0
        pltpu.sync_copy(acc_sh.at[pl.ds(r0, CKS)], val_v)
        pltpu.sync_copy(val_v, out_hbm.at[pl.ds(o0, CKS)])
        rem = TROWS - CKS
        pltpu.sync_copy(acc_sh.at[pl.ds(r0 + CKS, rem)], val_v.at[pl.ds(0, rem)])
        pltpu.sync_copy(val_v.at[pl.ds(0, rem)], out_hbm.at[pl.ds(o0 + CKS, rem)])

    return k(u, vids)


def _fin_a_body(a_ref, hg_ref, wf1a, wf1b, bf1, y_ref, st_ref):
    i = pl.program_id(0)
    y = (jnp.dot(a_ref[...], wf1a[...], preferred_element_type=jnp.float32)
         + jnp.dot(hg_ref[...], wf1b[...], preferred_element_type=jnp.float32)
         + bf1[...])
    y_ref[...] = y

    @pl.when(i == 0)
    def _():
        st_ref[...] = jnp.zeros_like(st_ref)

    st_ref[0:1, :] += jnp.sum(y, axis=0, keepdims=True)
    st_ref[1:2, :] += jnp.sum(y * y, axis=0, keepdims=True)


def _fin_a(a, hg, wf1a, wf1b, bf1):
    return pl.pallas_call(
        _fin_a_body,
        grid=(N_VERTS // BR,),
        in_specs=[
            pl.BlockSpec((BR, H), lambda i: (i, 0)),
            pl.BlockSpec((BR, H), lambda i: (i, 0)),
            pl.BlockSpec((H, H), lambda i: (0, 0)),
            pl.BlockSpec((H, H), lambda i: (0, 0)),
            pl.BlockSpec((1, H), lambda i: (0, 0)),
        ],
        out_specs=[
            pl.BlockSpec((BR, H), lambda i: (i, 0)),
            pl.BlockSpec((8, H), lambda i: (0, 0)),
        ],
        out_shape=(
            jax.ShapeDtypeStruct((N_VERTS, H), jnp.float32),
            jax.ShapeDtypeStruct((8, H), jnp.float32),
        ),
    )(a, hg, wf1a, wf1b, bf1)


def kernel(graph_x, node_pos, surface_x, verts, vnormals, vert_nbr_dist,
           nbr_vids, vert_nbr_ind, W_chem1, b_chem1, W_chem2, b_chem2,
           W_surf1, b_surf1, W_surf2, b_surf2, W_geom1, b_geom1, W_geom2,
           b_geom2, W_feat1, b_feat1, W_feat2, b_feat2):
    f32 = jnp.float32
    ind = vert_nbr_ind.astype(jnp.int32)
    vids = nbr_vids.astype(jnp.int32)

    wp = W_surf1[0:128]
    w1da = W_surf1[128:144]
    b1 = b_surf1.reshape(1, H)
    bc1 = b_chem1.reshape(1, H)
    bc2 = b_chem2.reshape(1, H)
    bg1 = b_geom1.reshape(1, H)
    bg2 = b_geom2.reshape(1, H)
    b2 = b_surf2.reshape(1, 2 * H)
    w2f = W_surf2[:, 0:H]
    w2c = W_surf2[:, H:2 * H]
    b2f = b_surf2[0:H].reshape(1, H)
    b2c = b_surf2[H:2 * H].reshape(1, H)
    wf1a = W_feat1[0:H]
    wf1b = W_feat1[H:2 * H]
    bf1 = b_feat1.reshape(1, H)
    bf2 = b_feat2.reshape(1, H)

    zv = jnp.zeros((N_VERTS, 5), f32)
    tvv = jnp.concatenate([verts, zv, vnormals, zv], axis=1)

    chem_out, tp = _chem_stage(graph_x, node_pos, W_chem1, bc1, W_chem2, bc2, wp)

    yg, stg1 = _lin1_stage(surface_x, W_geom1, bg1)
    stg2 = _mid_stage(yg, stg1, W_geom2, bg2)
    hg = _apply_stage(yg, stg1, stg2, W_geom2, bg2)

    gath_p, gath_vv = _gather_sc(tp, tvv, ind, vids)

    dist2d = vert_nbr_dist.reshape(E, 1)
    y, st1 = _edge_a(gath_p, gath_vv, dist2d, w1da, b1)
    st2 = _mid_stage(y, st1, W_surf2, b2)
    u = _edge_c(y, st1, st2, w2f, b2f, w2c, b2c)

    agg_full = _scatter_sc(u, vids)
    agg = agg_full[0:N_VERTS]

    yf, stf1 = _fin_a(agg, hg, wf1a, wf1b, bf1)
    stf2 = _mid_stage(yf, stf1, W_feat2, bf2)
    h_geom = _apply_stage(yf, stf1, stf2, W_feat2, bf2)

    return (h_geom, chem_out)


# speedup vs baseline: 1.0124x; 1.0124x over previous
import functools

import jax
import jax.numpy as jnp
from jax import lax
from jax.experimental import pallas as pl
from jax.experimental.pallas import tpu as pltpu
from jax.experimental.pallas import tpu_sc as plsc

H = 32
N_NODES = 10000
N_VERTS = 50000
E = 320000

NC = 2
NS = 16
NW = NC * NS
EW = E // NW
CKG = 1000
NCHG = EW // CKG

VH = 25088
ACC_ROWS = VH + 16
TROWS = VH // NS
ESUB = E // NS
CKS = 800
NCH_S = ESUB // CKS

BR = 5000
BE = 8000


def _rows_block(n):
    return BE if n % BE == 0 else BR


def _bn(y):
    m = jnp.mean(y, axis=0, keepdims=True)
    v = jnp.mean((y - m) * (y - m), axis=0, keepdims=True)
    return (y - m) * lax.rsqrt(v + 1e-5)


def _gdf(x, start, stop):
    step = (stop - start) / 7.0
    c = start + step * lax.broadcasted_iota(jnp.int32, (1, 8), 1).astype(jnp.float32)
    d = x - c
    return jnp.exp(-(d * d) / (step * step))


def _silu(x):
    return x * jax.nn.sigmoid(x)


def _chem_body(gx_ref, np_ref, wc1, bc1, wc2, bc2, wp, chem_ref, tp_ref):
    gx = gx_ref[...]
    h = _silu(_bn(jnp.dot(gx, wc1[...], preferred_element_type=jnp.float32) + bc1[...]))
    chem_ref[...] = _bn(jnp.dot(h, wc2[...], preferred_element_type=jnp.float32) + bc2[...])
    proj = jnp.dot(gx, wp[...], preferred_element_type=jnp.float32)
    pad = jnp.zeros((N_NODES, 13), jnp.float32)
    tp_ref[...] = jnp.concatenate([proj, np_ref[...], pad], axis=1)


def _chem_stage(graph_x, node_pos, wc1, bc1, wc2, bc2, wp):
    return pl.pallas_call(
        _chem_body,
        out_shape=(
            jax.ShapeDtypeStruct((N_NODES, H), jnp.float32),
            jax.ShapeDtypeStruct((N_NODES, 48), jnp.float32),
        ),
    )(graph_x, node_pos, wc1, bc1, wc2, bc2, wp)


def _lin1_body(x_ref, w1, b1, y_ref, st_ref):
    i = pl.program_id(0)
    y = jnp.dot(x_ref[...], w1[...], preferred_element_type=jnp.float32) + b1[...]
    y_ref[...] = y

    @pl.when(i == 0)
    def _():
        st_ref[...] = jnp.zeros_like(st_ref)

    st_ref[0:1, :] += jnp.sum(y, axis=0, keepdims=True)
    st_ref[1:2, :] += jnp.sum(y * y, axis=0, keepdims=True)


def _lin1_stage(x, w1, b1):
    n, fi = x.shape
    return pl.pallas_call(
        _lin1_body,
        grid=(n // BR,),
        in_specs=[
            pl.BlockSpec((BR, fi), lambda i: (i, 0)),
            pl.BlockSpec((fi, H), lambda i: (0, 0)),
            pl.BlockSpec((1, H), lambda i: (0, 0)),
        ],
        out_specs=[
            pl.BlockSpec((BR, H), lambda i: (i, 0)),
            pl.BlockSpec((8, H), lambda i: (0, 0)),
        ],
        out_shape=(
            jax.ShapeDtypeStruct((n, H), jnp.float32),
            jax.ShapeDtypeStruct((8, H), jnp.float32),
        ),
    )(x, w1, b1)


NB = N_VERTS // BR


def _vmlp2_body(two_in, x1_ref, *refs):
    if two_in:
        x2_ref, w1a, w1b, b1, w2, b2, out_ref, ysc, st1, st2 = refs
    else:
        w1a, b1, w2, b2, out_ref, ysc, st1, st2 = refs
    i = pl.program_id(0)

    @pl.when(i < NB)
    def _():
        y = jnp.dot(x1_ref[...], w1a[...], preferred_element_type=jnp.float32)
        if two_in:
            y = y + jnp.dot(x2_ref[...], w1b[...],
                            preferred_element_type=jnp.float32)
        y = y + b1[...]
        ysc[pl.ds(i * BR, BR), :] = y

        @pl.when(i == 0)
        def _():
            st1[...] = jnp.zeros_like(st1)

        st1[0:1, :] += jnp.sum(y, axis=0, keepdims=True)
        st1[1:2, :] += jnp.sum(y * y, axis=0, keepdims=True)

    @pl.when((i >= NB) & (i < 2 * NB))
    def _():
        j = i - NB
        y = ysc[pl.ds(j * BR, BR), :]
        mu1, is1 = _st_moments(st1, N_VERTS)
        h = _silu((y - mu1) * is1)
        z = jnp.dot(h, w2[...], preferred_element_type=jnp.float32) + b2[...]

        @pl.when(i == NB)
        def _():
            st2[...] = jnp.zeros_like(st2)

        st2[0:1, :] += jnp.sum(z, axis=0, keepdims=True)
        st2[1:2, :] += jnp.sum(z * z, axis=0, keepdims=True)

    @pl.when(i >= 2 * NB)
    def _():
        j = i - 2 * NB
        y = ysc[pl.ds(j * BR, BR), :]
        mu1, is1 = _st_moments(st1, N_VERTS)
        mu2, is2 = _st_moments(st2, N_VERTS)
        h = _silu((y - mu1) * is1)
        z = jnp.dot(h, w2[...] * is2, preferred_element_type=jnp.float32)
        out_ref[...] = z + (b2[...] - mu2) * is2


def _vmlp2_stage(x1, x2, w1a, w1b, b1, w2, b2):
    two_in = x2 is not None
    f1 = x1.shape[1]
    last = NB - 1
    in_specs = [pl.BlockSpec((BR, f1), lambda i: (jnp.minimum(i, last), 0))]
    args = [x1]
    if two_in:
        f2 = x2.shape[1]
        in_specs.append(pl.BlockSpec((BR, f2), lambda i: (jnp.minimum(i, last), 0)))
        args.append(x2)
    in_specs.append(pl.BlockSpec((f1, H), lambda i: (0, 0)))
    args.append(w1a)
    if two_in:
        in_specs.append(pl.BlockSpec((f2, H), lambda i: (0, 0)))
        args.append(w1b)
    in_specs += [
        pl.BlockSpec((1, H), lambda i: (0, 0)),
        pl.BlockSpec((H, H), lambda i: (0, 0)),
        pl.BlockSpec((1, H), lambda i: (0, 0)),
    ]
    args += [b1, w2, b2]
    return pl.pallas_call(
        functools.partial(_vmlp2_body, two_in),
        grid=(3 * NB,),
        in_specs=in_specs,
        out_specs=pl.BlockSpec((BR, H), lambda i: (jnp.maximum(i - 2 * NB, 0), 0)),
        out_shape=jax.ShapeDtypeStruct((N_VERTS, H), jnp.float32),
        scratch_shapes=[
            pltpu.VMEM((N_VERTS, H), jnp.float32),
            pltpu.VMEM((8, H), jnp.float32),
            pltpu.VMEM((8, H), jnp.float32),
        ],
    )(*args)


def _st_moments(st_ref, n):
    mu = st_ref[0:1, :] * (1.0 / n)
    var = st_ref[1:2, :] * (1.0 / n) - mu * mu
    return mu, lax.rsqrt(var + 1e-5)


def _mid_body(n, y_ref, st1, w2, b2, st_ref):
    i = pl.program_id(0)
    mu1, is1 = _st_moments(st1, n)
    h = _silu((y_ref[...] - mu1) * is1)
    z = jnp.dot(h, w2[...], preferred_element_type=jnp.float32) + b2[...]

    @pl.when(i == 0)
    def _():
        st_ref[...] = jnp.zeros_like(st_ref)

    st_ref[0:1, :] += jnp.sum(z, axis=0, keepdims=True)
    st_ref[1:2, :] += jnp.sum(z * z, axis=0, keepdims=True)


def _mid_stage(y, st1, w2, b2):
    n, _ = y.shape
    wo = w2.shape[1]
    br = _rows_block(n)
    return pl.pallas_call(
        functools.partial(_mid_body, n),
        grid=(n // br,),
        in_specs=[
            pl.BlockSpec((br, H), lambda i: (i, 0)),
            pl.BlockSpec((8, H), lambda i: (0, 0)),
            pl.BlockSpec((H, wo), lambda i: (0, 0)),
            pl.BlockSpec((1, wo), lambda i: (0, 0)),
        ],
        out_specs=pl.BlockSpec((8, wo), lambda i: (0, 0)),
        out_shape=jax.ShapeDtypeStruct((8, wo), jnp.float32),
    )(y, st1, w2, b2)


def _apply_body(n, y_ref, st1, st2, w2, b2, out_ref):
    mu1, is1 = _st_moments(st1, n)
    mu2, is2 = _st_moments(st2, n)
    h = _silu((y_ref[...] - mu1) * is1)
    z = jnp.dot(h, w2[...] * is2, preferred_element_type=jnp.float32)
    out_ref[...] = z + (b2[...] - mu2) * is2


def _apply_stage(y, st1, st2, w2, b2):
    n, _ = y.shape
    br = _rows_block(n)
    return pl.pallas_call(
        functools.partial(_apply_body, n),
        grid=(n // br,),
        in_specs=[
            pl.BlockSpec((br, H), lambda i: (i, 0)),
            pl.BlockSpec((8, H), lambda i: (0, 0)),
            pl.BlockSpec((8, H), lambda i: (0, 0)),
            pl.BlockSpec((H, H), lambda i: (0, 0)),
            pl.BlockSpec((1, H), lambda i: (0, 0)),
        ],
        out_specs=pl.BlockSpec((br, H), lambda i: (i, 0)),
        out_shape=jax.ShapeDtypeStruct((n, H), jnp.float32),
    )(y, st1, st2, w2, b2)


def _gather_sc(tp, tvv, ind, vids):
    mesh = plsc.VectorSubcoreMesh(core_axis_name="c", subcore_axis_name="s")

    @functools.partial(
        pl.kernel,
        out_type=(
            jax.ShapeDtypeStruct((E, 48), jnp.float32),
            jax.ShapeDtypeStruct((E, 16), jnp.float32),
        ),
        mesh=mesh,
        scratch_types=[
            pltpu.VMEM((CKG,), jnp.int32),
            pltpu.VMEM((CKG,), jnp.int32),
            pltpu.VMEM((CKG, 48), jnp.float32),
            pltpu.VMEM((CKG, 16), jnp.float32),
            pltpu.SemaphoreType.DMA,
            pltpu.SemaphoreType.DMA,
        ],
        compiler_params=pltpu.CompilerParams(use_tc_tiling_on_sc=False),
    )
    def k(tp_hbm, tvv_hbm, ind_hbm, vids_hbm, oa_hbm, oc_hbm,
          idx1_v, idx2_v, ra_v, rc_v, sem_a, sem_c):
        wid = lax.axis_index("s") * NC + lax.axis_index("c")

        def chunk(i, carry):
            base = pl.multiple_of(wid * EW + i * CKG, 8)
            pltpu.sync_copy(ind_hbm.at[pl.ds(base, CKG)], idx1_v)
            pltpu.sync_copy(vids_hbm.at[pl.ds(base, CKG)], idx2_v)
            da = pltpu.async_copy(tp_hbm.at[idx1_v], ra_v, sem_a)
            dc = pltpu.async_copy(tvv_hbm.at[idx2_v], rc_v, sem_c)
            da.wait()
            dc.wait()
            pltpu.sync_copy(ra_v, oa_hbm.at[pl.ds(base, CKG)])
            pltpu.sync_copy(rc_v, oc_hbm.at[pl.ds(base, CKG)])
            return carry

        lax.fori_loop(0, NCHG, chunk, 0)

    return k(tp, tvv, ind, vids)


def _edge_a_body(ga_ref, gvv_ref, d_ref, w1da, b1, y_ref, st_ref):
    i = pl.program_id(0)
    p8 = ga_ref[:, 32:40]
    v8 = gvv_ref[:, 0:8]
    n8 = gvv_ref[:, 8:16]
    dlt = p8 - v8
    ones8 = jnp.full((8, 1), 1.0, jnp.float32)
    nrm2 = jnp.dot(dlt * dlt, ones8, preferred_element_type=jnp.float32)
    dot = jnp.dot(dlt * n8, ones8, preferred_element_type=jnp.float32)
    ang = dot * lax.rsqrt(nrm2)
    gda = jnp.concatenate([_gdf(d_ref[...], 0.0, 8.0), _gdf(ang, -1.0, 1.0)],
                          axis=1)
    y = (ga_ref[:, 0:32]
         + jnp.dot(gda, w1da[...], preferred_element_type=jnp.float32)
         + b1[...])
    y_ref[...] = y

    @pl.when(i == 0)
    def _():
        st_ref[...] = jnp.zeros_like(st_ref)

    st_ref[0:1, :] += jnp.sum(y, axis=0, keepdims=True)
    st_ref[1:2, :] += jnp.sum(y * y, axis=0, keepdims=True)


def _edge_a(gath_p, gath_vv, dist2d, w1da, b1):
    return pl.pallas_call(
        _edge_a_body,
        grid=(E // BE,),
        in_specs=[
            pl.BlockSpec((BE, 48), lambda i: (i, 0)),
            pl.BlockSpec((BE, 16), lambda i: (i, 0)),
            pl.BlockSpec((BE, 1), lambda i: (i, 0)),
            pl.BlockSpec((16, H), lambda i: (0, 0)),
            pl.BlockSpec((1, H), lambda i: (0, 0)),
        ],
        out_specs=[
            pl.BlockSpec((BE, H), lambda i: (i, 0)),
            pl.BlockSpec((8, H), lambda i: (0, 0)),
        ],
        out_shape=(
            jax.ShapeDtypeStruct((E, H), jnp.float32),
            jax.ShapeDtypeStruct((8, H), jnp.float32),
        ),
    )(gath_p, gath_vv, dist2d, w1da, b1)


def _edge_c_body(y_ref, st1, st2, w2f, b2f, w2c, b2c, u_ref):
    mu1, is1 = _st_moments(st1, E)
    mu2, is2 = _st_moments(st2, E)
    mu2f, mu2c = mu2[:, 0:H], mu2[:, H:2 * H]
    is2f, is2c = is2[:, 0:H], is2[:, H:2 * H]
    h = _silu((y_ref[...] - mu1) * is1)
    zf = (jnp.dot(h, w2f[...] * is2f, preferred_element_type=jnp.float32)
          + (b2f[...] - mu2f) * is2f)
    zc = (jnp.dot(h, w2c[...] * is2c, preferred_element_type=jnp.float32)
          + (b2c[...] - mu2c) * is2c)
    gate = jax.nn.sigmoid(zf)
    sp = jnp.maximum(zc, 0.0) + jnp.log1p(jnp.exp(-jnp.abs(zc)))
    u_ref[...] = gate * sp


def _edge_c(y, st1, st2, w2f, b2f, w2c, b2c):
    return pl.pallas_call(
        _edge_c_body,
        grid=(E // BE,),
        in_specs=[
            pl.BlockSpec((BE, H), lambda i: (i, 0)),
            pl.BlockSpec((8, H), lambda i: (0, 0)),
            pl.BlockSpec((8, 2 * H), lambda i: (0, 0)),
            pl.BlockSpec((H, H), lambda i: (0, 0)),
            pl.BlockSpec((1, H), lambda i: (0, 0)),
            pl.BlockSpec((H, H), lambda i: (0, 0)),
            pl.BlockSpec((1, H), lambda i: (0, 0)),
        ],
        out_specs=pl.BlockSpec((BE, H), lambda i: (i, 0)),
        out_shape=jax.ShapeDtypeStruct((E, H), jnp.float32),
    )(y, st1, st2, w2f, b2f, w2c, b2c)


def _scatter_sc(u, vids):
    mesh = plsc.VectorSubcoreMesh(core_axis_name="c", subcore_axis_name="s")

    @functools.partial(
        pl.kernel,
        out_type=jax.ShapeDtypeStruct((NC * VH, H), jnp.float32),
        mesh=mesh,
        scratch_types=[
            pltpu.VMEM((CKS, H), jnp.float32),
            pltpu.VMEM((CKS,), jnp.int32),
            pltpu.VMEM_SHARED((ACC_ROWS, H), jnp.float32),
        ],
        compiler_params=pltpu.CompilerParams(use_tc_tiling_on_sc=False),
    )
    def k(u_hbm, vids_hbm, out_hbm, val_v, idx_v, acc_sh):
        cid = lax.axis_index("c")
        sid = lax.axis_index("s")
        r0 = sid * TROWS
        vbase = cid * VH

        def zbody(r, carry):
            z16 = jnp.zeros((16,), jnp.float32)
            val_v[r, 0:16] = z16
            val_v[r, 16:32] = z16
            return carry

        lax.fori_loop(0, CKS, zbody, 0)
        pltpu.sync_copy(val_v, acc_sh.at[pl.ds(r0, CKS)])
        pltpu.sync_copy(val_v.at[pl.ds(0, TROWS - CKS)],
                        acc_sh.at[pl.ds(r0 + CKS, TROWS - CKS)])

        @pl.when(sid == 0)
        def _():
            pltpu.sync_copy(val_v.at[pl.ds(0, 16)], acc_sh.at[pl.ds(VH, 16)])

        plsc.subcore_barrier()

        def chunk(i, carry):
            base = pl.multiple_of(sid * ESUB + i * CKS, 8)
            pltpu.sync_copy(u_hbm.at[pl.ds(base, CKS)], val_v)
            pltpu.sync_copy(vids_hbm.at[pl.ds(base, CKS)], idx_v)

            def xbody(j, c2):
                v = idx_v[pl.ds(j * 16, 16)] - vbase
                ok = (v >= 0) & (v < VH)
                idx_v[pl.ds(j * 16, 16)] = jnp.where(ok, v, VH)
                return c2

            lax.fori_loop(0, CKS // 16, xbody, 0)
            pltpu.sync_copy(val_v, acc_sh.at[idx_v], add=True)
            return carry

        lax.fori_loop(0, NCH_S, chunk, 0)
        plsc.subcore_barrier()

        o0 = vbase + r0
        pltpu.sync_copy(acc_sh.at[pl.ds(r0, CKS)], val_v)
        pltpu.sync_copy(val_v, out_hbm.at[pl.ds(o0, CKS)])
        rem = TROWS - CKS
        pltpu.sync_copy(acc_sh.at[pl.ds(r0 + CKS, rem)], val_v.at[pl.ds(0, rem)])
        pltpu.sync_copy(val_v.at[pl.ds(0, rem)], out_hbm.at[pl.ds(o0 + CKS, rem)])

    return k(u, vids)


def _fin_a_body(a_ref, hg_ref, wf1a, wf1b, bf1, y_ref, st_ref):
    i = pl.program_id(0)
    y = (jnp.dot(a_ref[...], wf1a[...], preferred_element_type=jnp.float32)
         + jnp.dot(hg_ref[...], wf1b[...], preferred_element_type=jnp.float32)
         + bf1[...])
    y_ref[...] = y

    @pl.when(i == 0)
    def _():
        st_ref[...] = jnp.zeros_like(st_ref)

    st_ref[0:1, :] += jnp.sum(y, axis=0, keepdims=True)
    st_ref[1:2, :] += jnp.sum(y * y, axis=0, keepdims=True)


def _fin_a(a, hg, wf1a, wf1b, bf1):
    return pl.pallas_call(
        _fin_a_body,
        grid=(N_VERTS // BR,),
        in_specs=[
            pl.BlockSpec((BR, H), lambda i: (i, 0)),
            pl.BlockSpec((BR, H), lambda i: (i, 0)),
            pl.BlockSpec((H, H), lambda i: (0, 0)),
            pl.BlockSpec((H, H), lambda i: (0, 0)),
            pl.BlockSpec((1, H), lambda i: (0, 0)),
        ],
        out_specs=[
            pl.BlockSpec((BR, H), lambda i: (i, 0)),
            pl.BlockSpec((8, H), lambda i: (0, 0)),
        ],
        out_shape=(
            jax.ShapeDtypeStruct((N_VERTS, H), jnp.float32),
            jax.ShapeDtypeStruct((8, H), jnp.float32),
        ),
    )(a, hg, wf1a, wf1b, bf1)


def kernel(graph_x, node_pos, surface_x, verts, vnormals, vert_nbr_dist,
           nbr_vids, vert_nbr_ind, W_chem1, b_chem1, W_chem2, b_chem2,
           W_surf1, b_surf1, W_surf2, b_surf2, W_geom1, b_geom1, W_geom2,
           b_geom2, W_feat1, b_feat1, W_feat2, b_feat2):
    f32 = jnp.float32
    ind = vert_nbr_ind.astype(jnp.int32)
    vids = nbr_vids.astype(jnp.int32)

    wp = W_surf1[0:128]
    w1da = W_surf1[128:144]
    b1 = b_surf1.reshape(1, H)
    bc1 = b_chem1.reshape(1, H)
    bc2 = b_chem2.reshape(1, H)
    bg1 = b_geom1.reshape(1, H)
    bg2 = b_geom2.reshape(1, H)
    b2 = b_surf2.reshape(1, 2 * H)
    w2f = W_surf2[:, 0:H]
    w2c = W_surf2[:, H:2 * H]
    b2f = b_surf2[0:H].reshape(1, H)
    b2c = b_surf2[H:2 * H].reshape(1, H)
    wf1a = W_feat1[0:H]
    wf1b = W_feat1[H:2 * H]
    bf1 = b_feat1.reshape(1, H)
    bf2 = b_feat2.reshape(1, H)

    zv = jnp.zeros((N_VERTS, 5), f32)
    tvv = jnp.concatenate([verts, zv, vnormals, zv], axis=1)

    chem_out, tp = _chem_stage(graph_x, node_pos, W_chem1, bc1, W_chem2, bc2, wp)

    hg = _vmlp2_stage(surface_x, None, W_geom1, None, bg1, W_geom2, bg2)

    gath_p, gath_vv = _gather_sc(tp, tvv, ind, vids)

    dist2d = vert_nbr_dist.reshape(E, 1)
    y, st1 = _edge_a(gath_p, gath_vv, dist2d, w1da, b1)
    st2 = _mid_stage(y, st1, W_surf2, b2)
    u = _edge_c(y, st1, st2, w2f, b2f, w2c, b2c)

    agg_full = _scatter_sc(u, vids)
    agg = agg_full[0:N_VERTS]

    h_geom = _vmlp2_stage(agg, hg, wf1a, wf1b, bf1, W_feat2, bf2)

    return (h_geom, chem_out)


# speedup vs baseline: 1.1248x; 1.1111x over previous
import functools

import jax
import jax.numpy as jnp
from jax import lax
from jax.experimental import pallas as pl
from jax.experimental.pallas import tpu as pltpu
from jax.experimental.pallas import tpu_sc as plsc

H = 32
N_NODES = 10000
N_VERTS = 50000
E = 320000

NC = 2
NS = 16
NW = NC * NS
EW = E // NW
CKG = 1000
NCHG = EW // CKG

VH = 25088
ACC_ROWS = VH + 16
TROWS = VH // NS
ESUB = E // NS
CKS = 800
NCH_S = ESUB // CKS

BR = 5000
BE = 8000


def _rows_block(n):
    return BE if n % BE == 0 else BR


def _bn(y):
    m = jnp.mean(y, axis=0, keepdims=True)
    v = jnp.mean((y - m) * (y - m), axis=0, keepdims=True)
    return (y - m) * lax.rsqrt(v + 1e-5)


def _gdf(x, start, stop):
    step = (stop - start) / 7.0
    c = start + step * lax.broadcasted_iota(jnp.int32, (1, 8), 1).astype(jnp.float32)
    d = x - c
    return jnp.exp(-(d * d) / (step * step))


def _silu(x):
    return x * jax.nn.sigmoid(x)


def _chem_body(gx_ref, np_ref, wc1, bc1, wc2, bc2, wp, chem_ref, tp_ref):
    gx = gx_ref[...]
    h = _silu(_bn(jnp.dot(gx, wc1[...], preferred_element_type=jnp.float32) + bc1[...]))
    chem_ref[...] = _bn(jnp.dot(h, wc2[...], preferred_element_type=jnp.float32) + bc2[...])
    proj = jnp.dot(gx, wp[...], preferred_element_type=jnp.float32)
    pad = jnp.zeros((N_NODES, 13), jnp.float32)
    tp_ref[...] = jnp.concatenate([proj, np_ref[...], pad], axis=1)


def _chem_stage(graph_x, node_pos, wc1, bc1, wc2, bc2, wp):
    return pl.pallas_call(
        _chem_body,
        out_shape=(
            jax.ShapeDtypeStruct((N_NODES, H), jnp.float32),
            jax.ShapeDtypeStruct((N_NODES, 48), jnp.float32),
        ),
    )(graph_x, node_pos, wc1, bc1, wc2, bc2, wp)


def _lin1_body(x_ref, w1, b1, y_ref, st_ref):
    i = pl.program_id(0)
    y = jnp.dot(x_ref[...], w1[...], preferred_element_type=jnp.float32) + b1[...]
    y_ref[...] = y

    @pl.when(i == 0)
    def _():
        st_ref[...] = jnp.zeros_like(st_ref)

    st_ref[0:1, :] += jnp.sum(y, axis=0, keepdims=True)
    st_ref[1:2, :] += jnp.sum(y * y, axis=0, keepdims=True)


def _lin1_stage(x, w1, b1):
    n, fi = x.shape
    return pl.pallas_call(
        _lin1_body,
        grid=(n // BR,),
        in_specs=[
            pl.BlockSpec((BR, fi), lambda i: (i, 0)),
            pl.BlockSpec((fi, H), lambda i: (0, 0)),
            pl.BlockSpec((1, H), lambda i: (0, 0)),
        ],
        out_specs=[
            pl.BlockSpec((BR, H), lambda i: (i, 0)),
            pl.BlockSpec((8, H), lambda i: (0, 0)),
        ],
        out_shape=(
            jax.ShapeDtypeStruct((n, H), jnp.float32),
            jax.ShapeDtypeStruct((8, H), jnp.float32),
        ),
    )(x, w1, b1)


NB = N_VERTS // BR


def _vmlp2_body(two_in, x1_ref, *refs):
    if two_in:
        x2_ref, w1a, w1b, b1, w2, b2, out_ref, ysc, st1, st2 = refs
    else:
        w1a, b1, w2, b2, out_ref, ysc, st1, st2 = refs
    i = pl.program_id(0)

    @pl.when(i < NB)
    def _():
        y = jnp.dot(x1_ref[...], w1a[...], preferred_element_type=jnp.float32)
        if two_in:
            y = y + jnp.dot(x2_ref[...], w1b[...],
                            preferred_element_type=jnp.float32)
        y = y + b1[...]
        ysc[pl.ds(i * BR, BR), :] = y

        @pl.when(i == 0)
        def _():
            st1[...] = jnp.zeros_like(st1)

        st1[0:1, :] += jnp.sum(y, axis=0, keepdims=True)
        st1[1:2, :] += jnp.sum(y * y, axis=0, keepdims=True)

    @pl.when((i >= NB) & (i < 2 * NB))
    def _():
        j = i - NB
        y = ysc[pl.ds(j * BR, BR), :]
        mu1, is1 = _st_moments(st1, N_VERTS)
        h = _silu((y - mu1) * is1)
        z = jnp.dot(h, w2[...], preferred_element_type=jnp.float32) + b2[...]

        @pl.when(i == NB)
        def _():
            st2[...] = jnp.zeros_like(st2)

        st2[0:1, :] += jnp.sum(z, axis=0, keepdims=True)
        st2[1:2, :] += jnp.sum(z * z, axis=0, keepdims=True)

    @pl.when(i >= 2 * NB)
    def _():
        j = i - 2 * NB
        y = ysc[pl.ds(j * BR, BR), :]
        mu1, is1 = _st_moments(st1, N_VERTS)
        mu2, is2 = _st_moments(st2, N_VERTS)
        h = _silu((y - mu1) * is1)
        z = jnp.dot(h, w2[...] * is2, preferred_element_type=jnp.float32)
        out_ref[...] = z + (b2[...] - mu2) * is2


def _vmlp2_stage(x1, x2, w1a, w1b, b1, w2, b2):
    two_in = x2 is not None
    f1 = x1.shape[1]
    last = NB - 1
    in_specs = [pl.BlockSpec((BR, f1), lambda i: (jnp.minimum(i, last), 0))]
    args = [x1]
    if two_in:
        f2 = x2.shape[1]
        in_specs.append(pl.BlockSpec((BR, f2), lambda i: (jnp.minimum(i, last), 0)))
        args.append(x2)
    in_specs.append(pl.BlockSpec((f1, H), lambda i: (0, 0)))
    args.append(w1a)
    if two_in:
        in_specs.append(pl.BlockSpec((f2, H), lambda i: (0, 0)))
        args.append(w1b)
    in_specs += [
        pl.BlockSpec((1, H), lambda i: (0, 0)),
        pl.BlockSpec((H, H), lambda i: (0, 0)),
        pl.BlockSpec((1, H), lambda i: (0, 0)),
    ]
    args += [b1, w2, b2]
    return pl.pallas_call(
        functools.partial(_vmlp2_body, two_in),
        grid=(3 * NB,),
        in_specs=in_specs,
        out_specs=pl.BlockSpec((BR, H), lambda i: (jnp.maximum(i - 2 * NB, 0), 0)),
        out_shape=jax.ShapeDtypeStruct((N_VERTS, H), jnp.float32),
        scratch_shapes=[
            pltpu.VMEM((N_VERTS, H), jnp.float32),
            pltpu.VMEM((8, H), jnp.float32),
            pltpu.VMEM((8, H), jnp.float32),
        ],
    )(*args)


def _st_moments(st_ref, n):
    mu = st_ref[0:1, :] * (1.0 / n)
    var = st_ref[1:2, :] * (1.0 / n) - mu * mu
    return mu, lax.rsqrt(var + 1e-5)


def _mid_body(n, y_ref, st1, w2, b2, st_ref):
    i = pl.program_id(0)
    mu1, is1 = _st_moments(st1, n)
    h = _silu((y_ref[...] - mu1) * is1)
    z = jnp.dot(h, w2[...], preferred_element_type=jnp.float32) + b2[...]

    @pl.when(i == 0)
    def _():
        st_ref[...] = jnp.zeros_like(st_ref)

    st_ref[0:1, :] += jnp.sum(z, axis=0, keepdims=True)
    st_ref[1:2, :] += jnp.sum(z * z, axis=0, keepdims=True)


def _mid_stage(y, st1, w2, b2):
    n, _ = y.shape
    wo = w2.shape[1]
    br = _rows_block(n)
    return pl.pallas_call(
        functools.partial(_mid_body, n),
        grid=(n // br,),
        in_specs=[
            pl.BlockSpec((br, H), lambda i: (i, 0)),
            pl.BlockSpec((8, H), lambda i: (0, 0)),
            pl.BlockSpec((H, wo), lambda i: (0, 0)),
            pl.BlockSpec((1, wo), lambda i: (0, 0)),
        ],
        out_specs=pl.BlockSpec((8, wo), lambda i: (0, 0)),
        out_shape=jax.ShapeDtypeStruct((8, wo), jnp.float32),
    )(y, st1, w2, b2)


def _apply_body(n, y_ref, st1, st2, w2, b2, out_ref):
    mu1, is1 = _st_moments(st1, n)
    mu2, is2 = _st_moments(st2, n)
    h = _silu((y_ref[...] - mu1) * is1)
    z = jnp.dot(h, w2[...] * is2, preferred_element_type=jnp.float32)
    out_ref[...] = z + (b2[...] - mu2) * is2


def _apply_stage(y, st1, st2, w2, b2):
    n, _ = y.shape
    br = _rows_block(n)
    return pl.pallas_call(
        functools.partial(_apply_body, n),
        grid=(n // br,),
        in_specs=[
            pl.BlockSpec((br, H), lambda i: (i, 0)),
            pl.BlockSpec((8, H), lambda i: (0, 0)),
            pl.BlockSpec((8, H), lambda i: (0, 0)),
            pl.BlockSpec((H, H), lambda i: (0, 0)),
            pl.BlockSpec((1, H), lambda i: (0, 0)),
        ],
        out_specs=pl.BlockSpec((br, H), lambda i: (i, 0)),
        out_shape=jax.ShapeDtypeStruct((n, H), jnp.float32),
    )(y, st1, st2, w2, b2)


def _gather_sc(tp, tvv, ind, vids):
    mesh = plsc.VectorSubcoreMesh(core_axis_name="c", subcore_axis_name="s")

    @functools.partial(
        pl.kernel,
        out_type=(
            jax.ShapeDtypeStruct((E, 48), jnp.float32),
            jax.ShapeDtypeStruct((E, 16), jnp.float32),
        ),
        mesh=mesh,
        scratch_types=[
            pltpu.VMEM((CKG,), jnp.int32),
            pltpu.VMEM((CKG,), jnp.int32),
            pltpu.VMEM((CKG, 48), jnp.float32),
            pltpu.VMEM((CKG, 16), jnp.float32),
            pltpu.SemaphoreType.DMA,
            pltpu.SemaphoreType.DMA,
        ],
        compiler_params=pltpu.CompilerParams(use_tc_tiling_on_sc=False),
    )
    def k(tp_hbm, tvv_hbm, ind_hbm, vids_hbm, oa_hbm, oc_hbm,
          idx1_v, idx2_v, ra_v, rc_v, sem_a, sem_c):
        wid = lax.axis_index("s") * NC + lax.axis_index("c")

        def chunk(i, carry):
            base = pl.multiple_of(wid * EW + i * CKG, 8)
            pltpu.sync_copy(ind_hbm.at[pl.ds(base, CKG)], idx1_v)
            pltpu.sync_copy(vids_hbm.at[pl.ds(base, CKG)], idx2_v)
            da = pltpu.async_copy(tp_hbm.at[idx1_v], ra_v, sem_a)
            dc = pltpu.async_copy(tvv_hbm.at[idx2_v], rc_v, sem_c)
            da.wait()
            dc.wait()
            pltpu.sync_copy(ra_v, oa_hbm.at[pl.ds(base, CKG)])
            pltpu.sync_copy(rc_v, oc_hbm.at[pl.ds(base, CKG)])
            return carry

        lax.fori_loop(0, NCHG, chunk, 0)

    return k(tp, tvv, ind, vids)


def _edge_a_body(ga_ref, gvv_ref, d_ref, w1da, b1, y_ref, st_ref):
    i = pl.program_id(0)
    p8 = ga_ref[:, 32:40]
    v8 = gvv_ref[:, 0:8]
    n8 = gvv_ref[:, 8:16]
    dlt = p8 - v8
    ones8 = jnp.full((8, 1), 1.0, jnp.float32)
    nrm2 = jnp.dot(dlt * dlt, ones8, preferred_element_type=jnp.float32)
    dot = jnp.dot(dlt * n8, ones8, preferred_element_type=jnp.float32)
    ang = dot * lax.rsqrt(nrm2)
    gda = jnp.concatenate([_gdf(d_ref[...], 0.0, 8.0), _gdf(ang, -1.0, 1.0)],
                          axis=1)
    y = (ga_ref[:, 0:32]
         + jnp.dot(gda, w1da[...], preferred_element_type=jnp.float32)
         + b1[...])
    y_ref[...] = y

    @pl.when(i == 0)
    def _():
        st_ref[...] = jnp.zeros_like(st_ref)

    st_ref[0:1, :] += jnp.sum(y, axis=0, keepdims=True)
    st_ref[1:2, :] += jnp.sum(y * y, axis=0, keepdims=True)


def _edge_a(gath_p, gath_vv, dist2d, w1da, b1):
    return pl.pallas_call(
        _edge_a_body,
        grid=(E // BE,),
        in_specs=[
            pl.BlockSpec((BE, 48), lambda i: (i, 0)),
            pl.BlockSpec((BE, 16), lambda i: (i, 0)),
            pl.BlockSpec((BE, 1), lambda i: (i, 0)),
            pl.BlockSpec((16, H), lambda i: (0, 0)),
            pl.BlockSpec((1, H), lambda i: (0, 0)),
        ],
        out_specs=[
            pl.BlockSpec((BE, H), lambda i: (i, 0)),
            pl.BlockSpec((8, H), lambda i: (0, 0)),
        ],
        out_shape=(
            jax.ShapeDtypeStruct((E, H), jnp.float32),
            jax.ShapeDtypeStruct((8, H), jnp.float32),
        ),
    )(gath_p, gath_vv, dist2d, w1da, b1)


def _tile4(v):
    return jnp.concatenate([v, v, v, v], axis=1)


E4 = E // 4
BE4 = BE // 4


def _emid_body(y_ref, st1, w2bd, b2, st_ref):
    i = pl.program_id(0)
    mu1, is1 = _st_moments(st1, E)
    h = _silu((y_ref[...] - _tile4(mu1)) * _tile4(is1))
    z = (jnp.dot(h, w2bd[...], preferred_element_type=jnp.float32)
         + _tile4(b2[...]))

    @pl.when(i == 0)
    def _():
        st_ref[...] = jnp.zeros_like(st_ref)

    st_ref[0:1, :] += jnp.sum(z, axis=0, keepdims=True)
    st_ref[1:2, :] += jnp.sum(z * z, axis=0, keepdims=True)


def _emid_stage(y4, st1, w2bd, b2):
    return pl.pallas_call(
        _emid_body,
        grid=(E4 // BE4,),
        in_specs=[
            pl.BlockSpec((BE4, 128), lambda i: (i, 0)),
            pl.BlockSpec((8, H), lambda i: (0, 0)),
            pl.BlockSpec((128, 256), lambda i: (0, 0)),
            pl.BlockSpec((1, 2 * H), lambda i: (0, 0)),
        ],
        out_specs=pl.BlockSpec((8, 256), lambda i: (0, 0)),
        out_shape=jax.ShapeDtypeStruct((8, 256), jnp.float32),
    )(y4, st1, w2bd, b2)


def _edge_c_body(y_ref, st1, st2p, wfbd, bf, wcbd, bc, u_ref):
    st2 = (st2p[:, 0:64] + st2p[:, 64:128] + st2p[:, 128:192]
           + st2p[:, 192:256])
    mu1, is1 = _st_moments(st1, E)
    mu2, is2 = _st_moments(st2, E)
    mu2f, mu2c = mu2[:, 0:H], mu2[:, H:2 * H]
    is2f, is2c = is2[:, 0:H], is2[:, H:2 * H]
    h = _silu((y_ref[...] - _tile4(mu1)) * _tile4(is1))
    isft = _tile4(is2f)
    isct = _tile4(is2c)
    zf = (jnp.dot(h, wfbd[...] * isft, preferred_element_type=jnp.float32)
          + (_tile4(bf[...] - mu2f)) * isft)
    zc = (jnp.dot(h, wcbd[...] * isct, preferred_element_type=jnp.float32)
          + (_tile4(bc[...] - mu2c)) * isct)
    gate = jax.nn.sigmoid(zf)
    sp = jnp.maximum(zc, 0.0) + jnp.log1p(jnp.exp(-jnp.abs(zc)))
    u_ref[...] = gate * sp


def _edge_c(y4, st1, st2p, wfbd, bf, wcbd, bc):
    return pl.pallas_call(
        _edge_c_body,
        grid=(E4 // BE4,),
        in_specs=[
            pl.BlockSpec((BE4, 128), lambda i: (i, 0)),
            pl.BlockSpec((8, H), lambda i: (0, 0)),
            pl.BlockSpec((8, 256), lambda i: (0, 0)),
            pl.BlockSpec((128, 128), lambda i: (0, 0)),
            pl.BlockSpec((1, H), lambda i: (0, 0)),
            pl.BlockSpec((128, 128), lambda i: (0, 0)),
            pl.BlockSpec((1, H), lambda i: (0, 0)),
        ],
        out_specs=pl.BlockSpec((BE4, 128), lambda i: (i, 0)),
        out_shape=jax.ShapeDtypeStruct((E4, 128), jnp.float32),
    )(y4, st1, st2p, wfbd, bf, wcbd, bc)


def _scatter_sc(u, vids):
    mesh = plsc.VectorSubcoreMesh(core_axis_name="c", subcore_axis_name="s")

    @functools.partial(
        pl.kernel,
        out_type=jax.ShapeDtypeStruct((NC * VH, H), jnp.float32),
        mesh=mesh,
        scratch_types=[
            pltpu.VMEM((CKS, H), jnp.float32),
            pltpu.VMEM((CKS,), jnp.int32),
            pltpu.VMEM_SHARED((ACC_ROWS, H), jnp.float32),
        ],
        compiler_params=pltpu.CompilerParams(use_tc_tiling_on_sc=False),
    )
    def k(u_hbm, vids_hbm, out_hbm, val_v, idx_v, acc_sh):
        cid = lax.axis_index("c")
        sid = lax.axis_index("s")
        r0 = sid * TROWS
        vbase = cid * VH

        def zbody(r, carry):
            z16 = jnp.zeros((16,), jnp.float32)
            val_v[r, 0:16] = z16
            val_v[r, 16:32] = z16
            return carry

        lax.fori_loop(0, CKS, zbody, 0)
        pltpu.sync_copy(val_v, acc_sh.at[pl.ds(r0, CKS)])
        pltpu.sync_copy(val_v.at[pl.ds(0, TROWS - CKS)],
                        acc_sh.at[pl.ds(r0 + CKS, TROWS - CKS)])

        @pl.when(sid == 0)
        def _():
            pltpu.sync_copy(val_v.at[pl.ds(0, 16)], acc_sh.at[pl.ds(VH, 16)])

        plsc.subcore_barrier()

        def chunk(i, carry):
            base = pl.multiple_of(sid * ESUB + i * CKS, 8)
            pltpu.sync_copy(u_hbm.at[pl.ds(base, CKS)], val_v)
            pltpu.sync_copy(vids_hbm.at[pl.ds(base, CKS)], idx_v)

            def xbody(j, c2):
                v = idx_v[pl.ds(j * 16, 16)] - vbase
                ok = (v >= 0) & (v < VH)
                idx_v[pl.ds(j * 16, 16)] = jnp.where(ok, v, VH)
                return c2

            lax.fori_loop(0, CKS // 16, xbody, 0)
            pltpu.sync_copy(val_v, acc_sh.at[idx_v], add=True)
            return carry

        lax.fori_loop(0, NCH_S, chunk, 0)
        plsc.subcore_barrier()

        o0 = vbase + r0
        pltpu.sync_copy(acc_sh.at[pl.ds(r0, CKS)], val_v)
        pltpu.sync_copy(val_v, out_hbm.at[pl.ds(o0, CKS)])
        rem = TROWS - CKS
        pltpu.sync_copy(acc_sh.at[pl.ds(r0 + CKS, rem)], val_v.at[pl.ds(0, rem)])
        pltpu.sync_copy(val_v.at[pl.ds(0, rem)], out_hbm.at[pl.ds(o0 + CKS, rem)])

    return k(u, vids)


def _fin_a_body(a_ref, hg_ref, wf1a, wf1b, bf1, y_ref, st_ref):
    i = pl.program_id(0)
    y = (jnp.dot(a_ref[...], wf1a[...], preferred_element_type=jnp.float32)
         + jnp.dot(hg_ref[...], wf1b[...], preferred_element_type=jnp.float32)
         + bf1[...])
    y_ref[...] = y

    @pl.when(i == 0)
    def _():
        st_ref[...] = jnp.zeros_like(st_ref)

    st_ref[0:1, :] += jnp.sum(y, axis=0, keepdims=True)
    st_ref[1:2, :] += jnp.sum(y * y, axis=0, keepdims=True)


def _fin_a(a, hg, wf1a, wf1b, bf1):
    return pl.pallas_call(
        _fin_a_body,
        grid=(N_VERTS // BR,),
        in_specs=[
            pl.BlockSpec((BR, H), lambda i: (i, 0)),
            pl.BlockSpec((BR, H), lambda i: (i, 0)),
            pl.BlockSpec((H, H), lambda i: (0, 0)),
            pl.BlockSpec((H, H), lambda i: (0, 0)),
            pl.BlockSpec((1, H), lambda i: (0, 0)),
        ],
        out_specs=[
            pl.BlockSpec((BR, H), lambda i: (i, 0)),
            pl.BlockSpec((8, H), lambda i: (0, 0)),
        ],
        out_shape=(
            jax.ShapeDtypeStruct((N_VERTS, H), jnp.float32),
            jax.ShapeDtypeStruct((8, H), jnp.float32),
        ),
    )(a, hg, wf1a, wf1b, bf1)


def kernel(graph_x, node_pos, surface_x, verts, vnormals, vert_nbr_dist,
           nbr_vids, vert_nbr_ind, W_chem1, b_chem1, W_chem2, b_chem2,
           W_surf1, b_surf1, W_surf2, b_surf2, W_geom1, b_geom1, W_geom2,
           b_geom2, W_feat1, b_feat1, W_feat2, b_feat2):
    f32 = jnp.float32
    ind = vert_nbr_ind.astype(jnp.int32)
    vids = nbr_vids.astype(jnp.int32)

    wp = W_surf1[0:128]
    w1da = W_surf1[128:144]
    b1 = b_surf1.reshape(1, H)
    bc1 = b_chem1.reshape(1, H)
    bc2 = b_chem2.reshape(1, H)
    bg1 = b_geom1.reshape(1, H)
    bg2 = b_geom2.reshape(1, H)
    b2 = b_surf2.reshape(1, 2 * H)
    w2f = W_surf2[:, 0:H]
    w2c = W_surf2[:, H:2 * H]
    b2f = b_surf2[0:H].reshape(1, H)
    b2c = b_surf2[H:2 * H].reshape(1, H)
    wf1a = W_feat1[0:H]
    wf1b = W_feat1[H:2 * H]
    bf1 = b_feat1.reshape(1, H)
    bf2 = b_feat2.reshape(1, H)

    zv = jnp.zeros((N_VERTS, 5), f32)
    tvv = jnp.concatenate([verts, zv, vnormals, zv], axis=1)

    chem_out, tp = _chem_stage(graph_x, node_pos, W_chem1, bc1, W_chem2, bc2, wp)

    hg = _vmlp2_stage(surface_x, None, W_geom1, None, bg1, W_geom2, bg2)

    gath_p, gath_vv = _gather_sc(tp, tvv, ind, vids)

    dist2d = vert_nbr_dist.reshape(E, 1)
    y, st1 = _edge_a(gath_p, gath_vv, dist2d, w1da, b1)

    y4 = y.reshape(E4, 128)
    eye4 = jnp.eye(4, dtype=f32)
    w2bd = jnp.kron(eye4, W_surf2)
    wfbd = jnp.kron(eye4, w2f)
    wcbd = jnp.kron(eye4, w2c)
    st2p = _emid_stage(y4, st1, w2bd, b2)
    u4 = _edge_c(y4, st1, st2p, wfbd, b2f, wcbd, b2c)

    agg_full = _scatter_sc(u4.reshape(E, H), vids)
    agg = agg_full[0:N_VERTS]

    h_geom = _vmlp2_stage(agg, hg, wf1a, wf1b, bf1, W_feat2, bf2)

    return (h_geom, chem_out)


# speedup vs baseline: 1.4113x; 1.2546x over previous
import functools

import jax
import jax.numpy as jnp
from jax import lax
from jax.experimental import pallas as pl
from jax.experimental.pallas import tpu as pltpu
from jax.experimental.pallas import tpu_sc as plsc

H = 32
N_NODES = 10000
N_VERTS = 50000
E = 320000

NC = 2
NS = 16
NW = NC * NS
EW = E // NW
CKG = 400
NCHG = EW // CKG

VH = 25088
ACC_ROWS = VH + 16
TROWS = VH // NS
ESUB = E // NS
CKS = 800
NCH_S = ESUB // CKS

BR = 5000
BE = 8000


def _rows_block(n):
    return BE if n % BE == 0 else BR


def _bn(y):
    m = jnp.mean(y, axis=0, keepdims=True)
    v = jnp.mean((y - m) * (y - m), axis=0, keepdims=True)
    return (y - m) * lax.rsqrt(v + 1e-5)


def _gdf(x, start, stop):
    step = (stop - start) / 7.0
    c = start + step * lax.broadcasted_iota(jnp.int32, (1, 8), 1).astype(jnp.float32)
    d = x - c
    return jnp.exp(-(d * d) / (step * step))


def _silu(x):
    return x * jax.nn.sigmoid(x)


def _chem_body(gx_ref, np_ref, wc1, bc1, wc2, bc2, wp, chem_ref, tp_ref):
    gx = gx_ref[...]
    h = _silu(_bn(jnp.dot(gx, wc1[...], preferred_element_type=jnp.float32) + bc1[...]))
    chem_ref[...] = _bn(jnp.dot(h, wc2[...], preferred_element_type=jnp.float32) + bc2[...])
    proj = jnp.dot(gx, wp[...], preferred_element_type=jnp.float32)
    pad = jnp.zeros((N_NODES, 13), jnp.float32)
    tp_ref[...] = jnp.concatenate([proj, np_ref[...], pad], axis=1)


def _chem_stage(graph_x, node_pos, wc1, bc1, wc2, bc2, wp):
    return pl.pallas_call(
        _chem_body,
        out_shape=(
            jax.ShapeDtypeStruct((N_NODES, H), jnp.float32),
            jax.ShapeDtypeStruct((N_NODES, 48), jnp.float32),
        ),
    )(graph_x, node_pos, wc1, bc1, wc2, bc2, wp)


def _lin1_body(x_ref, w1, b1, y_ref, st_ref):
    i = pl.program_id(0)
    y = jnp.dot(x_ref[...], w1[...], preferred_element_type=jnp.float32) + b1[...]
    y_ref[...] = y

    @pl.when(i == 0)
    def _():
        st_ref[...] = jnp.zeros_like(st_ref)

    st_ref[0:1, :] += jnp.sum(y, axis=0, keepdims=True)
    st_ref[1:2, :] += jnp.sum(y * y, axis=0, keepdims=True)


def _lin1_stage(x, w1, b1):
    n, fi = x.shape
    return pl.pallas_call(
        _lin1_body,
        grid=(n // BR,),
        in_specs=[
            pl.BlockSpec((BR, fi), lambda i: (i, 0)),
            pl.BlockSpec((fi, H), lambda i: (0, 0)),
            pl.BlockSpec((1, H), lambda i: (0, 0)),
        ],
        out_specs=[
            pl.BlockSpec((BR, H), lambda i: (i, 0)),
            pl.BlockSpec((8, H), lambda i: (0, 0)),
        ],
        out_shape=(
            jax.ShapeDtypeStruct((n, H), jnp.float32),
            jax.ShapeDtypeStruct((8, H), jnp.float32),
        ),
    )(x, w1, b1)


NB = N_VERTS // BR


def _vmlp2_body(two_in, x1_ref, *refs):
    if two_in:
        x2_ref, w1a, w1b, b1, w2, b2, out_ref, ysc, st1, st2 = refs
    else:
        w1a, b1, w2, b2, out_ref, ysc, st1, st2 = refs
    i = pl.program_id(0)

    @pl.when(i < NB)
    def _():
        y = jnp.dot(x1_ref[...], w1a[...], preferred_element_type=jnp.float32)
        if two_in:
            y = y + jnp.dot(x2_ref[...], w1b[...],
                            preferred_element_type=jnp.float32)
        y = y + b1[...]
        ysc[pl.ds(i * BR, BR), :] = y

        @pl.when(i == 0)
        def _():
            st1[...] = jnp.zeros_like(st1)

        st1[0:1, :] += jnp.sum(y, axis=0, keepdims=True)
        st1[1:2, :] += jnp.sum(y * y, axis=0, keepdims=True)

    @pl.when((i >= NB) & (i < 2 * NB))
    def _():
        j = i - NB
        y = ysc[pl.ds(j * BR, BR), :]
        mu1, is1 = _st_moments(st1, N_VERTS)
        h = _silu((y - mu1) * is1)
        z = jnp.dot(h, w2[...], preferred_element_type=jnp.float32) + b2[...]

        @pl.when(i == NB)
        def _():
            st2[...] = jnp.zeros_like(st2)

        st2[0:1, :] += jnp.sum(z, axis=0, keepdims=True)
        st2[1:2, :] += jnp.sum(z * z, axis=0, keepdims=True)

    @pl.when(i >= 2 * NB)
    def _():
        j = i - 2 * NB
        y = ysc[pl.ds(j * BR, BR), :]
        mu1, is1 = _st_moments(st1, N_VERTS)
        mu2, is2 = _st_moments(st2, N_VERTS)
        h = _silu((y - mu1) * is1)
        z = jnp.dot(h, w2[...] * is2, preferred_element_type=jnp.float32)
        out_ref[...] = z + (b2[...] - mu2) * is2


def _vmlp2_stage(x1, x2, w1a, w1b, b1, w2, b2):
    two_in = x2 is not None
    f1 = x1.shape[1]
    last = NB - 1
    in_specs = [pl.BlockSpec((BR, f1), lambda i: (jnp.minimum(i, last), 0))]
    args = [x1]
    if two_in:
        f2 = x2.shape[1]
        in_specs.append(pl.BlockSpec((BR, f2), lambda i: (jnp.minimum(i, last), 0)))
        args.append(x2)
    in_specs.append(pl.BlockSpec((f1, H), lambda i: (0, 0)))
    args.append(w1a)
    if two_in:
        in_specs.append(pl.BlockSpec((f2, H), lambda i: (0, 0)))
        args.append(w1b)
    in_specs += [
        pl.BlockSpec((1, H), lambda i: (0, 0)),
        pl.BlockSpec((H, H), lambda i: (0, 0)),
        pl.BlockSpec((1, H), lambda i: (0, 0)),
    ]
    args += [b1, w2, b2]
    return pl.pallas_call(
        functools.partial(_vmlp2_body, two_in),
        grid=(3 * NB,),
        in_specs=in_specs,
        out_specs=pl.BlockSpec((BR, H), lambda i: (jnp.maximum(i - 2 * NB, 0), 0)),
        out_shape=jax.ShapeDtypeStruct((N_VERTS, H), jnp.float32),
        scratch_shapes=[
            pltpu.VMEM((N_VERTS, H), jnp.float32),
            pltpu.VMEM((8, H), jnp.float32),
            pltpu.VMEM((8, H), jnp.float32),
        ],
    )(*args)


def _st_moments(st_ref, n):
    mu = st_ref[0:1, :] * (1.0 / n)
    var = st_ref[1:2, :] * (1.0 / n) - mu * mu
    return mu, lax.rsqrt(var + 1e-5)


def _mid_body(n, y_ref, st1, w2, b2, st_ref):
    i = pl.program_id(0)
    mu1, is1 = _st_moments(st1, n)
    h = _silu((y_ref[...] - mu1) * is1)
    z = jnp.dot(h, w2[...], preferred_element_type=jnp.float32) + b2[...]

    @pl.when(i == 0)
    def _():
        st_ref[...] = jnp.zeros_like(st_ref)

    st_ref[0:1, :] += jnp.sum(z, axis=0, keepdims=True)
    st_ref[1:2, :] += jnp.sum(z * z, axis=0, keepdims=True)


def _mid_stage(y, st1, w2, b2):
    n, _ = y.shape
    wo = w2.shape[1]
    br = _rows_block(n)
    return pl.pallas_call(
        functools.partial(_mid_body, n),
        grid=(n // br,),
        in_specs=[
            pl.BlockSpec((br, H), lambda i: (i, 0)),
            pl.BlockSpec((8, H), lambda i: (0, 0)),
            pl.BlockSpec((H, wo), lambda i: (0, 0)),
            pl.BlockSpec((1, wo), lambda i: (0, 0)),
        ],
        out_specs=pl.BlockSpec((8, wo), lambda i: (0, 0)),
        out_shape=jax.ShapeDtypeStruct((8, wo), jnp.float32),
    )(y, st1, w2, b2)


def _apply_body(n, y_ref, st1, st2, w2, b2, out_ref):
    mu1, is1 = _st_moments(st1, n)
    mu2, is2 = _st_moments(st2, n)
    h = _silu((y_ref[...] - mu1) * is1)
    z = jnp.dot(h, w2[...] * is2, preferred_element_type=jnp.float32)
    out_ref[...] = z + (b2[...] - mu2) * is2


def _apply_stage(y, st1, st2, w2, b2):
    n, _ = y.shape
    br = _rows_block(n)
    return pl.pallas_call(
        functools.partial(_apply_body, n),
        grid=(n // br,),
        in_specs=[
            pl.BlockSpec((br, H), lambda i: (i, 0)),
            pl.BlockSpec((8, H), lambda i: (0, 0)),
            pl.BlockSpec((8, H), lambda i: (0, 0)),
            pl.BlockSpec((H, H), lambda i: (0, 0)),
            pl.BlockSpec((1, H), lambda i: (0, 0)),
        ],
        out_specs=pl.BlockSpec((br, H), lambda i: (i, 0)),
        out_shape=jax.ShapeDtypeStruct((n, H), jnp.float32),
    )(y, st1, st2, w2, b2)


def _gather_sc(tp, tvv, ind, vids, dist):
    mesh = plsc.VectorSubcoreMesh(core_axis_name="c", subcore_axis_name="s")

    @functools.partial(
        pl.kernel,
        out_type=jax.ShapeDtypeStruct((E, 48), jnp.float32),
        mesh=mesh,
        scratch_types=[
            pltpu.VMEM((CKG,), jnp.int32),
            pltpu.VMEM((CKG,), jnp.int32),
            pltpu.VMEM((CKG,), jnp.float32),
            pltpu.VMEM((CKG, 48), jnp.float32),
            pltpu.VMEM((CKG, 16), jnp.float32),
            pltpu.SemaphoreType.DMA,
            pltpu.SemaphoreType.DMA,
        ],
        compiler_params=pltpu.CompilerParams(use_tc_tiling_on_sc=False,
                                             needs_layout_passes=False),
    )
    def k(tp_hbm, tvv_hbm, ind_hbm, vids_hbm, dist_hbm, oa_hbm,
          idx1_v, idx2_v, dist_v, ra_v, rc_v, sem_a, sem_c):
        wid = lax.axis_index("s") * NC + lax.axis_index("c")

        def chunk(i, carry):
            base = pl.multiple_of(wid * EW + i * CKG, 8)
            pltpu.sync_copy(ind_hbm.at[pl.ds(base, CKG)], idx1_v)
            pltpu.sync_copy(vids_hbm.at[pl.ds(base, CKG)], idx2_v)
            pltpu.sync_copy(dist_hbm.at[pl.ds(base, CKG)], dist_v)
            da = pltpu.async_copy(tp_hbm.at[idx1_v], ra_v, sem_a)
            dc = pltpu.async_copy(tvv_hbm.at[idx2_v], rc_v, sem_c)
            da.wait()
            dc.wait()

            def proc(j, c2):
                rows = lax.broadcasted_iota(jnp.int32, (16,), 0) + j * 16

                def col(ref, cc):
                    return plsc.load_gather(
                        ref, [rows, jnp.full((16,), cc, jnp.int32)])

                dx = col(ra_v, 32) - col(rc_v, 0)
                dy = col(ra_v, 33) - col(rc_v, 1)
                dz = col(ra_v, 34) - col(rc_v, 2)
                n2 = dx * dx + dy * dy + dz * dz
                dt = dx * col(rc_v, 8) + dy * col(rc_v, 9) + dz * col(rc_v, 10)
                xi = plsc.bitcast(n2, jnp.int32)
                r = plsc.bitcast(0x5F3759DF - (xi >> 1), jnp.float32)
                r = r * (1.5 - 0.5 * n2 * r * r)
                r = r * (1.5 - 0.5 * n2 * r * r)
                ang = dt * r
                plsc.store_scatter(
                    ra_v, [rows, jnp.full((16,), 35, jnp.int32)], ang)
                plsc.store_scatter(
                    ra_v, [rows, jnp.full((16,), 36, jnp.int32)],
                    dist_v[pl.ds(j * 16, 16)])
                return c2

            lax.fori_loop(0, CKG // 16, proc, 0)
            pltpu.sync_copy(ra_v, oa_hbm.at[pl.ds(base, CKG)])
            return carry

        lax.fori_loop(0, NCHG, chunk, 0)

    return k(tp, tvv, ind, vids, dist)


def _edge_a_body(ga_ref, w1da, b1, y_ref, st_ref):
    i = pl.program_id(0)
    ang = ga_ref[:, 35:36]
    d = ga_ref[:, 36:37]
    gda = jnp.concatenate([_gdf(d, 0.0, 8.0), _gdf(ang, -1.0, 1.0)], axis=1)
    y = (ga_ref[:, 0:32]
         + jnp.dot(gda, w1da[...], preferred_element_type=jnp.float32)
         + b1[...])
    y_ref[...] = y

    @pl.when(i == 0)
    def _():
        st_ref[...] = jnp.zeros_like(st_ref)

    st_ref[0:1, :] += jnp.sum(y, axis=0, keepdims=True)
    st_ref[1:2, :] += jnp.sum(y * y, axis=0, keepdims=True)


def _edge_a(gath_p, w1da, b1):
    return pl.pallas_call(
        _edge_a_body,
        grid=(E // BE,),
        in_specs=[
            pl.BlockSpec((BE, 48), lambda i: (i, 0)),
            pl.BlockSpec((16, H), lambda i: (0, 0)),
            pl.BlockSpec((1, H), lambda i: (0, 0)),
        ],
        out_specs=[
            pl.BlockSpec((BE, H), lambda i: (i, 0)),
            pl.BlockSpec((8, H), lambda i: (0, 0)),
        ],
        out_shape=(
            jax.ShapeDtypeStruct((E, H), jnp.float32),
            jax.ShapeDtypeStruct((8, H), jnp.float32),
        ),
    )(gath_p, w1da, b1)


def _tile4(v):
    return jnp.concatenate([v, v, v, v], axis=1)


E4 = E // 4
BE4 = BE // 4


def _emid_body(y_ref, st1, w2bd, b2, st_ref):
    i = pl.program_id(0)
    mu1, is1 = _st_moments(st1, E)
    h = _silu((y_ref[...] - _tile4(mu1)) * _tile4(is1))
    z = (jnp.dot(h, w2bd[...], preferred_element_type=jnp.float32)
         + _tile4(b2[...]))

    @pl.when(i == 0)
    def _():
        st_ref[...] = jnp.zeros_like(st_ref)

    st_ref[0:1, :] += jnp.sum(z, axis=0, keepdims=True)
    st_ref[1:2, :] += jnp.sum(z * z, axis=0, keepdims=True)


def _emid_stage(y4, st1, w2bd, b2):
    return pl.pallas_call(
        _emid_body,
        grid=(E4 // BE4,),
        in_specs=[
            pl.BlockSpec((BE4, 128), lambda i: (i, 0)),
            pl.BlockSpec((8, H), lambda i: (0, 0)),
            pl.BlockSpec((128, 256), lambda i: (0, 0)),
            pl.BlockSpec((1, 2 * H), lambda i: (0, 0)),
        ],
        out_specs=pl.BlockSpec((8, 256), lambda i: (0, 0)),
        out_shape=jax.ShapeDtypeStruct((8, 256), jnp.float32),
    )(y4, st1, w2bd, b2)


def _edge_c_body(y_ref, st1, st2p, wfbd, bf, wcbd, bc, u_ref):
    st2 = (st2p[:, 0:64] + st2p[:, 64:128] + st2p[:, 128:192]
           + st2p[:, 192:256])
    mu1, is1 = _st_moments(st1, E)
    mu2, is2 = _st_moments(st2, E)
    mu2f, mu2c = mu2[:, 0:H], mu2[:, H:2 * H]
    is2f, is2c = is2[:, 0:H], is2[:, H:2 * H]
    h = _silu((y_ref[...] - _tile4(mu1)) * _tile4(is1))
    isft = _tile4(is2f)
    isct = _tile4(is2c)
    zf = (jnp.dot(h, wfbd[...] * isft, preferred_element_type=jnp.float32)
          + (_tile4(bf[...] - mu2f)) * isft)
    zc = (jnp.dot(h, wcbd[...] * isct, preferred_element_type=jnp.float32)
          + (_tile4(bc[...] - mu2c)) * isct)
    gate = jax.nn.sigmoid(zf)
    sp = jnp.maximum(zc, 0.0) + jnp.log1p(jnp.exp(-jnp.abs(zc)))
    u_ref[...] = gate * sp


def _edge_c(y4, st1, st2p, wfbd, bf, wcbd, bc):
    return pl.pallas_call(
        _edge_c_body,
        grid=(E4 // BE4,),
        in_specs=[
            pl.BlockSpec((BE4, 128), lambda i: (i, 0)),
            pl.BlockSpec((8, H), lambda i: (0, 0)),
            pl.BlockSpec((8, 256), lambda i: (0, 0)),
            pl.BlockSpec((128, 128), lambda i: (0, 0)),
            pl.BlockSpec((1, H), lambda i: (0, 0)),
            pl.BlockSpec((128, 128), lambda i: (0, 0)),
            pl.BlockSpec((1, H), lambda i: (0, 0)),
        ],
        out_specs=pl.BlockSpec((BE4, 128), lambda i: (i, 0)),
        out_shape=jax.ShapeDtypeStruct((E4, 128), jnp.float32),
    )(y4, st1, st2p, wfbd, bf, wcbd, bc)


def _scatter_sc(u, vids):
    mesh = plsc.VectorSubcoreMesh(core_axis_name="c", subcore_axis_name="s")

    @functools.partial(
        pl.kernel,
        out_type=jax.ShapeDtypeStruct((NC * VH, H), jnp.float32),
        mesh=mesh,
        scratch_types=[
            pltpu.VMEM((CKS, H), jnp.float32),
            pltpu.VMEM((CKS,), jnp.int32),
            pltpu.VMEM_SHARED((ACC_ROWS, H), jnp.float32),
        ],
        compiler_params=pltpu.CompilerParams(use_tc_tiling_on_sc=False),
    )
    def k(u_hbm, vids_hbm, out_hbm, val_v, idx_v, acc_sh):
        cid = lax.axis_index("c")
        sid = lax.axis_index("s")
        r0 = sid * TROWS
        vbase = cid * VH

        def zbody(r, carry):
            z16 = jnp.zeros((16,), jnp.float32)
            val_v[r, 0:16] = z16
            val_v[r, 16:32] = z16
            return carry

        lax.fori_loop(0, CKS, zbody, 0)
        pltpu.sync_copy(val_v, acc_sh.at[pl.ds(r0, CKS)])
        pltpu.sync_copy(val_v.at[pl.ds(0, TROWS - CKS)],
                        acc_sh.at[pl.ds(r0 + CKS, TROWS - CKS)])

        @pl.when(sid == 0)
        def _():
            pltpu.sync_copy(val_v.at[pl.ds(0, 16)], acc_sh.at[pl.ds(VH, 16)])

        plsc.subcore_barrier()

        def chunk(i, carry):
            base = pl.multiple_of(sid * ESUB + i * CKS, 8)
            pltpu.sync_copy(u_hbm.at[pl.ds(base, CKS)], val_v)
            pltpu.sync_copy(vids_hbm.at[pl.ds(base, CKS)], idx_v)

            def xbody(j, c2):
                v = idx_v[pl.ds(j * 16, 16)] - vbase
                ok = (v >= 0) & (v < VH)
                idx_v[pl.ds(j * 16, 16)] = jnp.where(ok, v, VH)
                return c2

            lax.fori_loop(0, CKS // 16, xbody, 0)
            pltpu.sync_copy(val_v, acc_sh.at[idx_v], add=True)
            return carry

        lax.fori_loop(0, NCH_S, chunk, 0)
        plsc.subcore_barrier()

        o0 = vbase + r0
        pltpu.sync_copy(acc_sh.at[pl.ds(r0, CKS)], val_v)
        pltpu.sync_copy(val_v, out_hbm.at[pl.ds(o0, CKS)])
        rem = TROWS - CKS
        pltpu.sync_copy(acc_sh.at[pl.ds(r0 + CKS, rem)], val_v.at[pl.ds(0, rem)])
        pltpu.sync_copy(val_v.at[pl.ds(0, rem)], out_hbm.at[pl.ds(o0 + CKS, rem)])

    return k(u, vids)


def _fin_a_body(a_ref, hg_ref, wf1a, wf1b, bf1, y_ref, st_ref):
    i = pl.program_id(0)
    y = (jnp.dot(a_ref[...], wf1a[...], preferred_element_type=jnp.float32)
         + jnp.dot(hg_ref[...], wf1b[...], preferred_element_type=jnp.float32)
         + bf1[...])
    y_ref[...] = y

    @pl.when(i == 0)
    def _():
        st_ref[...] = jnp.zeros_like(st_ref)

    st_ref[0:1, :] += jnp.sum(y, axis=0, keepdims=True)
    st_ref[1:2, :] += jnp.sum(y * y, axis=0, keepdims=True)


def _fin_a(a, hg, wf1a, wf1b, bf1):
    return pl.pallas_call(
        _fin_a_body,
        grid=(N_VERTS // BR,),
        in_specs=[
            pl.BlockSpec((BR, H), lambda i: (i, 0)),
            pl.BlockSpec((BR, H), lambda i: (i, 0)),
            pl.BlockSpec((H, H), lambda i: (0, 0)),
            pl.BlockSpec((H, H), lambda i: (0, 0)),
            pl.BlockSpec((1, H), lambda i: (0, 0)),
        ],
        out_specs=[
            pl.BlockSpec((BR, H), lambda i: (i, 0)),
            pl.BlockSpec((8, H), lambda i: (0, 0)),
        ],
        out_shape=(
            jax.ShapeDtypeStruct((N_VERTS, H), jnp.float32),
            jax.ShapeDtypeStruct((8, H), jnp.float32),
        ),
    )(a, hg, wf1a, wf1b, bf1)


def kernel(graph_x, node_pos, surface_x, verts, vnormals, vert_nbr_dist,
           nbr_vids, vert_nbr_ind, W_chem1, b_chem1, W_chem2, b_chem2,
           W_surf1, b_surf1, W_surf2, b_surf2, W_geom1, b_geom1, W_geom2,
           b_geom2, W_feat1, b_feat1, W_feat2, b_feat2):
    f32 = jnp.float32
    ind = vert_nbr_ind.astype(jnp.int32)
    vids = nbr_vids.astype(jnp.int32)

    wp = W_surf1[0:128]
    w1da = W_surf1[128:144]
    b1 = b_surf1.reshape(1, H)
    bc1 = b_chem1.reshape(1, H)
    bc2 = b_chem2.reshape(1, H)
    bg1 = b_geom1.reshape(1, H)
    bg2 = b_geom2.reshape(1, H)
    b2 = b_surf2.reshape(1, 2 * H)
    w2f = W_surf2[:, 0:H]
    w2c = W_surf2[:, H:2 * H]
    b2f = b_surf2[0:H].reshape(1, H)
    b2c = b_surf2[H:2 * H].reshape(1, H)
    wf1a = W_feat1[0:H]
    wf1b = W_feat1[H:2 * H]
    bf1 = b_feat1.reshape(1, H)
    bf2 = b_feat2.reshape(1, H)

    zv = jnp.zeros((N_VERTS, 5), f32)
    tvv = jnp.concatenate([verts, zv, vnormals, zv], axis=1)

    chem_out, tp = _chem_stage(graph_x, node_pos, W_chem1, bc1, W_chem2, bc2, wp)

    hg = _vmlp2_stage(surface_x, None, W_geom1, None, bg1, W_geom2, bg2)

    gath_p = _gather_sc(tp, tvv, ind, vids, vert_nbr_dist)

    y, st1 = _edge_a(gath_p, w1da, b1)

    y4 = y.reshape(E4, 128)
    eye4 = jnp.eye(4, dtype=f32)
    w2bd = jnp.kron(eye4, W_surf2)
    wfbd = jnp.kron(eye4, w2f)
    wcbd = jnp.kron(eye4, w2c)
    st2p = _emid_stage(y4, st1, w2bd, b2)
    u4 = _edge_c(y4, st1, st2p, wfbd, b2f, wcbd, b2c)

    agg_full = _scatter_sc(u4.reshape(E, H), vids)
    agg = agg_full[0:N_VERTS]

    h_geom = _vmlp2_stage(agg, hg, wf1a, wf1b, bf1, W_feat2, bf2)

    return (h_geom, chem_out)
